# baseline (speedup 1.0000x reference)
import functools

import jax
import jax.numpy as jnp
from jax import lax
from jax.experimental import pallas as pl
from jax.experimental.pallas import tpu as pltpu
from jax.experimental.pallas import tpu_sc as plsc

_FIELD_DIM = 38462
_NUM_FIELDS = 26
_EMBED_DIM = 16
_TOTAL = _FIELD_DIM * _NUM_FIELDS

_BATCH = 4096
_NW = 32
_BPW = _BATCH // _NW
_NBLK = _BPW // 16


def _fm_body(xt_hbm, embf_hbm, lin_hbm, bias_hbm, out_hbm,
             idx_v, gidx, grows, lin_v, obuf, bias_v, sem, sem2):
    c = lax.axis_index("c")
    s = lax.axis_index("s")
    w = s * 2 + c

    pltpu.sync_copy(xt_hbm.at[:, pl.ds(w * _BPW, _BPW)], idx_v)
    pltpu.sync_copy(bias_hbm, bias_v)

    for j in range(_NUM_FIELDS):
        off = jnp.int32(j * _FIELD_DIM)
        for k in range(_NBLK):
            r = idx_v[j, pl.ds(k * 16, 16)] + off
            idx_v[j, pl.ds(k * 16, 16)] = r
            gidx[j, pl.ds(k * 16, 16)] = r

    lin_descs = [
        pltpu.async_copy(lin_hbm.at[idx_v.at[j]], lin_v.at[j], sem2)
        for j in range(_NUM_FIELDS)
    ]

    step = jnp.int32(_TOTAL)
    zeros = jnp.zeros((16,), jnp.float32)

    def gather_dim():
        descs = [
            pltpu.async_copy(embf_hbm.at[gidx.at[j]], grows.at[j], sem)
            for j in range(_NUM_FIELDS)
        ]
        for q in descs:
            q.wait()
        for j in range(_NUM_FIELDS):
            for k in range(_NBLK):
                gidx[j, pl.ds(k * 16, 16)] = gidx[j, pl.ds(k * 16, 16)] + step

    def dbody2(d, ytot):
        gather_dim()
        out = []
        for blk in range(_NBLK):
            s1 = zeros
            s2 = zeros
            for j in range(_NUM_FIELDS):
                v = grows[j, pl.ds(blk * 16, 16)]
                s1 = s1 + v
                s2 = s2 + v * v
            out.append(ytot[blk] + 0.5 * (s1 * s1 - s2))
        return tuple(out)

    def dbody3(d, ytot):
        gather_dim()
        out = []
        for blk in range(_NBLK):
            s1 = zeros
            s2 = zeros
            s3 = zeros
            for j in range(_NUM_FIELDS):
                v = grows[j, pl.ds(blk * 16, 16)]
                q = v * v
                s1 = s1 + v
                s2 = s2 + q
                s3 = s3 + q * v
            e3 = (s1 * s1 * s1 - 3.0 * s1 * s2 + 2.0 * s3) * (1.0 / 6.0)
            out.append(ytot[blk] + e3)
        return tuple(out)

    ytot = tuple(zeros for _ in range(_NBLK))
    ytot = lax.fori_loop(0, _EMBED_DIM, dbody2, ytot)
    ytot = lax.fori_loop(0, _EMBED_DIM, dbody3, ytot)

    for q in lin_descs:
        q.wait()

    bias16 = bias_v[...]
    for blk in range(_NBLK):
        acc = zeros
        for j in range(_NUM_FIELDS):
            acc = acc + lin_v[j, pl.ds(blk * 16, 16)]
        y = ytot[blk] + acc + bias16
        obuf[pl.ds(blk * 16, 16)] = 1.0 / (1.0 + jnp.exp(-y))

    pltpu.sync_copy(obuf, out_hbm.at[pl.ds(w * _BPW, _BPW)])


@jax.jit
def _fm_sc(xt, embf, lin1d, bias16):
    mesh = plsc.VectorSubcoreMesh(core_axis_name="c", subcore_axis_name="s")
    f = functools.partial(
        pl.kernel,
        mesh=mesh,
        out_type=jax.ShapeDtypeStruct((_BATCH,), jnp.float32),
        scratch_types=[
            pltpu.VMEM((_NUM_FIELDS, _BPW), jnp.int32),
            pltpu.VMEM((_NUM_FIELDS, _BPW), jnp.int32),
            pltpu.VMEM((_NUM_FIELDS, _BPW), jnp.float32),
            pltpu.VMEM((_NUM_FIELDS, _BPW), jnp.float32),
            pltpu.VMEM((_BPW,), jnp.float32),
            pltpu.VMEM((16,), jnp.float32),
            pltpu.SemaphoreType.DMA,
            pltpu.SemaphoreType.DMA,
        ],
        compiler_params=pltpu.CompilerParams(
            needs_layout_passes=False, use_tc_tiling_on_sc=False),
    )(_fm_body)
    return f(xt, embf, lin1d, bias16)


def kernel(x, emb_table, lin_table, bias):
    xt = x.astype(jnp.int32).T
    embf = emb_table.T.reshape(-1)
    lin1d = lin_table.T.reshape(-1)
    bias16 = jnp.broadcast_to(bias.astype(jnp.float32), (16,))
    return _fm_sc(xt, embf, lin1d, bias16)


# speedup vs baseline: 5.2507x; 5.2507x over previous
import functools

import jax
import jax.numpy as jnp
from jax import lax
from jax.experimental import pallas as pl
from jax.experimental.pallas import tpu as pltpu
from jax.experimental.pallas import tpu_sc as plsc

_FIELD_DIM = 38462
_NUM_FIELDS = 26
_EMBED_DIM = 16
_ROW = 2 * _EMBED_DIM
_TOTAL = _FIELD_DIM * _NUM_FIELDS

_BATCH = 4096
_NW = 32
_BPW = _BATCH // _NW
_BLK = 32
_NBLKS = _BPW // _BLK


def _fm_body(xt_hbm, emb_hbm, lin_hbm, bias_hbm, out_hbm,
             idx_v, lin_v, buf0, buf1, rbuf, obuf, bias_v,
             sem_lin, sem0, sem1):
    c = lax.axis_index("c")
    s = lax.axis_index("s")
    w = s * 2 + c

    pltpu.sync_copy(xt_hbm.at[:, pl.ds(w * _BPW, _BPW)], idx_v)
    pltpu.sync_copy(bias_hbm, bias_v)

    for j in range(_NUM_FIELDS):
        off = jnp.int32(j * _FIELD_DIM)
        for k in range(_BPW // 16):
            idx_v[j, pl.ds(k * 16, 16)] = idx_v[j, pl.ds(k * 16, 16)] + off

    lin_descs = [
        pltpu.async_copy(lin_hbm.at[idx_v.at[j]], lin_v.at[j], sem_lin)
        for j in range(_NUM_FIELDS)
    ]

    bufs = (buf0, buf1)
    sems = (sem0, sem1)

    def start_block(b):
        bb = bufs[b % 2]
        sm = sems[b % 2]
        return [
            pltpu.async_copy(
                emb_hbm.at[idx_v.at[j, pl.ds(b * _BLK, _BLK)]], bb.at[j], sm)
            for j in range(_NUM_FIELDS)
        ]

    zeros = jnp.zeros((16,), jnp.float32)
    lanes = lax.iota(jnp.int32, 16)
    pending = start_block(0)

    for b in range(_NBLKS):
        next_pending = start_block(b + 1) if b + 1 < _NBLKS else None
        for q in pending:
            q.wait()
        pending = next_pending
        bb = bufs[b % 2]

        def sbody(i, carry, bb=bb):
            s1lo = zeros
            s2lo = zeros
            s1 = zeros
            s2 = zeros
            s3 = zeros
            for j in range(_NUM_FIELDS):
                vlo = bb[j, i, pl.ds(0, 16)]
                vhi = bb[j, i, pl.ds(16, 16)]
                s1lo = s1lo + vlo
                s2lo = s2lo + vlo * vlo
                q2 = vhi * vhi
                s1 = s1 + vhi
                s2 = s2 + q2
                s3 = s3 + q2 * vhi
            e2 = 0.5 * (s1lo * s1lo - s2lo)
            e3 = (s1 * s1 * s1 - 3.0 * s1 * s2 + 2.0 * s3) * (1.0 / 6.0)
            rbuf[pl.ds(i * 16, 16)] = e2 + e3
            return carry

        lax.fori_loop(0, _BLK, sbody, 0)

        if b == 0:
            for q in lin_descs:
                q.wait()

        for ch in range(_BLK // 16):
            acc = zeros
            for d in range(16):
                acc = acc + plsc.load_gather(
                    rbuf, [lanes * 16 + jnp.int32(ch * 256 + d)])
            for j in range(_NUM_FIELDS):
                acc = acc + lin_v[j, pl.ds(b * _BLK + ch * 16, 16)]
            y = acc + bias_v[...]
            obuf[pl.ds(b * _BLK + ch * 16, 16)] = 1.0 / (1.0 + jnp.exp(-y))

    pltpu.sync_copy(obuf, out_hbm.at[pl.ds(w * _BPW, _BPW)])


@jax.jit
def _fm_sc(xt, emb, lin1d, bias16):
    mesh = plsc.VectorSubcoreMesh(core_axis_name="c", subcore_axis_name="s")
    f = functools.partial(
        pl.kernel,
        mesh=mesh,
        out_type=jax.ShapeDtypeStruct((_BATCH,), jnp.float32),
        scratch_types=[
            pltpu.VMEM((_NUM_FIELDS, _BPW), jnp.int32),
            pltpu.VMEM((_NUM_FIELDS, _BPW), jnp.float32),
            pltpu.VMEM((_NUM_FIELDS, _BLK, _ROW), jnp.float32),
            pltpu.VMEM((_NUM_FIELDS, _BLK, _ROW), jnp.float32),
            pltpu.VMEM((_BLK * 16,), jnp.float32),
            pltpu.VMEM((_BPW,), jnp.float32),
            pltpu.VMEM((16,), jnp.float32),
            pltpu.SemaphoreType.DMA,
            pltpu.SemaphoreType.DMA,
            pltpu.SemaphoreType.DMA,
        ],
        compiler_params=pltpu.CompilerParams(
            needs_layout_passes=False, use_tc_tiling_on_sc=False),
    )(_fm_body)
    return f(xt, emb, lin1d, bias16)


def kernel(x, emb_table, lin_table, bias):
    xt = x.astype(jnp.int32).T
    lin1d = lin_table.reshape(-1)
    bias16 = jnp.broadcast_to(bias.astype(jnp.float32), (16,))
    return _fm_sc(xt, emb_table, lin1d, bias16)
